# Initial kernel scaffold; baseline (speedup 1.0000x reference)
import jax
import jax.numpy as jnp
from jax.experimental import pallas as pl


def kernel(x, tgrid):
    raise NotImplementedError("write your pallas kernel here")


# speedup vs baseline: 10.8950x; 10.8950x over previous
import jax
import jax.numpy as jnp
from jax import lax
from jax.experimental import pallas as pl
from jax.experimental.pallas import tpu as pltpu
from jax.experimental.pallas import tpu_sc as plsc

RES_ = 512
FDIM_ = 32
NPTS = 1000000
OUTD = 3 + FDIM_

NC = 2
NS = 16
NW = NC * NS
LANES = 16

B = 80
NV = B // LANES
NTAP = 12
NCHUNKS = NPTS // B


def _tec_body(x_hbm, tab_hbm, out_hbm, xv, idxv, wv, rows, outv, sem):
  cid = lax.axis_index("c")
  sid = lax.axis_index("s")
  wid = sid * NC + cid
  count = (NCHUNKS - wid + NW - 1) // NW
  iot = lax.iota(jnp.int32, LANES)

  def chunk_body(g, _):
    ck = wid + g * NW
    base = ck * B
    pltpu.sync_copy(x_hbm.at[pl.ds(base * 3, B * 3)], xv)

    def idx_body(i, _):
      b = i * LANES
      bi = iot + b
      b3 = bi * 3
      xc = [plsc.load_gather(xv, [b3 + c]) for c in range(3)]
      for p, (ca, cb) in enumerate(((0, 1), (1, 2), (2, 0))):
        xf = (xc[ca] + 1.0) * 0.5 * (RES_ - 1)
        yf = (xc[cb] + 1.0) * 0.5 * (RES_ - 1)
        x0 = xf.astype(jnp.int32)
        y0 = yf.astype(jnp.int32)
        fx = xf - x0.astype(jnp.float32)
        fy = yf - y0.astype(jnp.float32)
        x1 = jnp.minimum(x0 + 1, RES_ - 1)
        y1 = jnp.minimum(y0 + 1, RES_ - 1)
        r0 = y0 * RES_ + (p * RES_ * RES_)
        r1 = y1 * RES_ + (p * RES_ * RES_)
        gx = 1.0 - fx
        gy = 1.0 - fy
        taps = ((r0 + x0, gy * gx), (r0 + x1, gy * fx),
                (r1 + x0, fy * gx), (r1 + x1, fy * fx))
        for t, (ti, tw) in enumerate(taps):
          idxv[p * 4 + t, pl.ds(b, LANES)] = ti
          wv[p * 4 + t, pl.ds(b, LANES)] = tw
      return 0

    lax.fori_loop(0, NV, idx_body, 0)

    cps = [pltpu.async_copy(tab_hbm.at[idxv.at[t]],
                            rows.at[pl.ds(t * B, B), :], sem)
           for t in range(NTAP)]
    for cp in cps:
      cp.wait()

    def acc_body(i, _):
      b = i * LANES
      bi = iot + b
      b3 = bi * 3
      b35 = bi * OUTD
      for c in range(3):
        plsc.store_scatter(outv, [b35 + c], plsc.load_gather(xv, [b3 + c]))
      ws = [wv[t, pl.ds(b, LANES)] for t in range(NTAP)]
      rvs = [bi + t * B for t in range(NTAP)]
      for ch in range(FDIM_):
        chs = jnp.full((LANES,), ch, jnp.int32)
        acc = plsc.load_gather(rows, [rvs[0], chs]) * ws[0]
        for t in range(1, NTAP):
          acc = acc + plsc.load_gather(rows, [rvs[t], chs]) * ws[t]
        plsc.store_scatter(outv, [b35 + (3 + ch)], acc)
      return 0

    lax.fori_loop(0, NV, acc_body, 0)
    pltpu.sync_copy(outv, out_hbm.at[pl.ds(base * OUTD, B * OUTD)])
    return 0

  lax.fori_loop(0, count, chunk_body, 0)


@jax.jit
def kernel(x, tgrid):
  tab = tgrid.reshape(3, FDIM_, RES_ * RES_).transpose(0, 2, 1)
  tab = tab.reshape(3 * RES_ * RES_, FDIM_)
  mesh = plsc.VectorSubcoreMesh(core_axis_name="c", subcore_axis_name="s",
                                num_cores=NC, num_subcores=NS)
  run = pl.kernel(
      _tec_body,
      out_type=jax.ShapeDtypeStruct((NPTS * OUTD,), jnp.float32),
      mesh=mesh,
      compiler_params=pltpu.CompilerParams(needs_layout_passes=False,
                                           use_tc_tiling_on_sc=False),
      scratch_types=[
          pltpu.VMEM((B * 3,), jnp.float32),
          pltpu.VMEM((NTAP, B), jnp.int32),
          pltpu.VMEM((NTAP, B), jnp.float32),
          pltpu.VMEM((NTAP * B, FDIM_), jnp.float32),
          pltpu.VMEM((B * OUTD,), jnp.float32),
          pltpu.SemaphoreType.DMA,
      ],
  )
  out_flat = run(x.reshape(NPTS * 3), tab)
  return out_flat.reshape(NPTS, OUTD)


# speedup vs baseline: 11.0172x; 1.0112x over previous
import jax
import jax.numpy as jnp
from jax import lax
from jax.experimental import pallas as pl
from jax.experimental.pallas import tpu as pltpu
from jax.experimental.pallas import tpu_sc as plsc

RES_ = 512
FDIM_ = 32
NPTS = 1000000
OUTD = 3 + FDIM_

NC = 2
NS = 16
NW = NC * NS
LANES = 16

B = 80
NV = B // LANES
NTAP = 12
NCHUNKS = NPTS // B


def _tec_body(x_hbm, tab_hbm, out_hbm, xv, idxv, wv, rows, outv, sem):
  cid = lax.axis_index("c")
  sid = lax.axis_index("s")
  wid = sid * NC + cid
  count = (NCHUNKS - wid + NW - 1) // NW
  iot = lax.iota(jnp.int32, LANES)

  def chunk_body(g, _):
    ck = wid + g * NW
    base = ck * B
    pltpu.sync_copy(x_hbm.at[pl.ds(base * 3, B * 3)], xv)

    def idx_body(i, _):
      b = i * LANES
      bi = iot + b
      b3 = bi * 3
      xc = [plsc.load_gather(xv, [b3 + c]) for c in range(3)]
      for p, (ca, cb) in enumerate(((0, 1), (1, 2), (2, 0))):
        xf = (xc[ca] + 1.0) * 0.5 * (RES_ - 1)
        yf = (xc[cb] + 1.0) * 0.5 * (RES_ - 1)
        x0 = xf.astype(jnp.int32)
        y0 = yf.astype(jnp.int32)
        fx = xf - x0.astype(jnp.float32)
        fy = yf - y0.astype(jnp.float32)
        x1 = jnp.minimum(x0 + 1, RES_ - 1)
        y1 = jnp.minimum(y0 + 1, RES_ - 1)
        r0 = y0 * RES_ + (p * RES_ * RES_)
        r1 = y1 * RES_ + (p * RES_ * RES_)
        gx = 1.0 - fx
        gy = 1.0 - fy
        taps = ((r0 + x0, gy * gx), (r0 + x1, gy * fx),
                (r1 + x0, fy * gx), (r1 + x1, fy * fx))
        for t, (ti, tw) in enumerate(taps):
          idxv[p * 4 + t, pl.ds(b, LANES)] = ti
          wv[p * 4 + t, pl.ds(b, LANES)] = tw
      return 0

    lax.fori_loop(0, NV, idx_body, 0)

    cps = [pltpu.async_copy(tab_hbm.at[idxv.at[t]],
                            rows.at[pl.ds(t * B, B), :], sem)
           for t in range(NTAP)]
    for cp in cps:
      cp.wait()

    def acc_body(i, _):
      b = i * LANES
      bi = iot + b
      b3 = bi * 3
      b35 = bi * OUTD
      for c in range(3):
        plsc.store_scatter(outv, [b35 + c], plsc.load_gather(xv, [b3 + c]))
      ws = [wv[t, pl.ds(b, LANES)] for t in range(NTAP)]
      rvs = [bi + t * B for t in range(NTAP)]
      for ch in range(FDIM_):
        chs = jnp.full((LANES,), ch, jnp.int32)
        acc = plsc.load_gather(rows, [rvs[0], chs]) * ws[0]
        for t in range(1, NTAP):
          acc = acc + plsc.load_gather(rows, [rvs[t], chs]) * ws[t]
        plsc.store_scatter(outv, [b35 + (3 + ch)], acc)
      return 0

    lax.fori_loop(0, NV, acc_body, 0)
    pltpu.sync_copy(outv, out_hbm.at[pl.ds(base * OUTD, B * OUTD)])
    return 0

  lax.fori_loop(0, count, chunk_body, 0)


def _transpose_body(g_ref, t_ref):
  t_ref[...] = g_ref[0].T


def _relayout(tgrid):
  S = RES_ * RES_
  CB = 8192
  g3 = tgrid.reshape(3, FDIM_, S)
  tab = pl.pallas_call(
      _transpose_body,
      grid=(3, S // CB),
      in_specs=[pl.BlockSpec((1, FDIM_, CB), lambda p, s: (p, 0, s))],
      out_specs=pl.BlockSpec((CB, FDIM_), lambda p, s: (p * (S // CB) + s, 0)),
      out_shape=jax.ShapeDtypeStruct((3 * S, FDIM_), jnp.float32),
  )(g3)
  return tab


@jax.jit
def kernel(x, tgrid):
  tab = _relayout(tgrid)
  mesh = plsc.VectorSubcoreMesh(core_axis_name="c", subcore_axis_name="s",
                                num_cores=NC, num_subcores=NS)
  run = pl.kernel(
      _tec_body,
      out_type=jax.ShapeDtypeStruct((NPTS * OUTD,), jnp.float32),
      mesh=mesh,
      compiler_params=pltpu.CompilerParams(needs_layout_passes=False,
                                           use_tc_tiling_on_sc=False),
      scratch_types=[
          pltpu.VMEM((B * 3,), jnp.float32),
          pltpu.VMEM((NTAP, B), jnp.int32),
          pltpu.VMEM((NTAP, B), jnp.float32),
          pltpu.VMEM((NTAP * B, FDIM_), jnp.float32),
          pltpu.VMEM((B * OUTD,), jnp.float32),
          pltpu.SemaphoreType.DMA,
      ],
  )
  out_flat = run(x.reshape(NPTS * 3), tab)
  return out_flat.reshape(NPTS, OUTD)


# speedup vs baseline: 11.1076x; 1.0082x over previous
import jax
import jax.numpy as jnp
from jax import lax
from jax.experimental import pallas as pl
from jax.experimental.pallas import tpu as pltpu
from jax.experimental.pallas import tpu_sc as plsc

RES_ = 512
FDIM_ = 32
NPTS = 1000000
OUTD = 3 + FDIM_
QROW = 4 * FDIM_

NC = 2
NS = 16
NW = NC * NS
LANES = 16

B = 80
NV = B // LANES
NPL = 3
NCHUNKS = NPTS // B


def _quad_body(a_ref, b_ref, t_ref):
  y = pl.program_id(1)
  a = a_ref[0, :, y % 8, :]
  b = b_ref[0, :, jnp.minimum(y + 1, RES_ - 1) % 8, :]
  a1 = jnp.concatenate([a[:, 1:], a[:, RES_ - 1:]], axis=1)
  b1 = jnp.concatenate([b[:, 1:], b[:, RES_ - 1:]], axis=1)
  t_ref[...] = jnp.concatenate([a.T, a1.T, b.T, b1.T], axis=1)


def _build_quad_table(tgrid):
  g4 = tgrid.reshape(3, FDIM_, RES_, RES_)
  line = pl.BlockSpec((1, FDIM_, 8, RES_), lambda p, y: (p, 0, y // 8, 0))
  line_n = pl.BlockSpec((1, FDIM_, 8, RES_),
                        lambda p, y: (p, 0, jnp.minimum(y + 1, RES_ - 1) // 8, 0))
  return pl.pallas_call(
      _quad_body,
      grid=(3, RES_),
      in_specs=[line, line_n],
      out_specs=pl.BlockSpec((RES_, QROW), lambda p, y: (p * RES_ + y, 0)),
      out_shape=jax.ShapeDtypeStruct((3 * RES_ * RES_, QROW), jnp.float32),
  )(g4, g4)


def _tec_body(x_hbm, tab_hbm, out_hbm, xv, idxv, wv, rows, outv, sem):
  cid = lax.axis_index("c")
  sid = lax.axis_index("s")
  wid = sid * NC + cid
  count = (NCHUNKS - wid + NW - 1) // NW
  iot = lax.iota(jnp.int32, LANES)

  def chunk_body(g, _):
    ck = wid + g * NW
    base = ck * B
    pltpu.sync_copy(x_hbm.at[pl.ds(base * 3, B * 3)], xv)

    def idx_body(i, _):
      b = i * LANES
      bi = iot + b
      b3 = bi * 3
      xc = [plsc.load_gather(xv, [b3 + c]) for c in range(3)]
      for p, (ca, cb) in enumerate(((0, 1), (1, 2), (2, 0))):
        xf = (xc[ca] + 1.0) * 0.5 * (RES_ - 1)
        yf = (xc[cb] + 1.0) * 0.5 * (RES_ - 1)
        x0 = xf.astype(jnp.int32)
        y0 = yf.astype(jnp.int32)
        fx = xf - x0.astype(jnp.float32)
        fy = yf - y0.astype(jnp.float32)
        gx = 1.0 - fx
        gy = 1.0 - fy
        idxv[p, pl.ds(b, LANES)] = y0 * RES_ + x0 + (p * RES_ * RES_)
        for q, tw in enumerate((gy * gx, gy * fx, fy * gx, fy * fx)):
          wv[p * 4 + q, pl.ds(b, LANES)] = tw
      return 0

    lax.fori_loop(0, NV, idx_body, 0)

    cps = [pltpu.async_copy(tab_hbm.at[idxv.at[p]],
                            rows.at[pl.ds(p * B, B), :], sem)
           for p in range(NPL)]
    for cp in cps:
      cp.wait()

    def acc_body(i, _):
      b = i * LANES
      bi = iot + b
      b3 = bi * 3
      b35 = bi * OUTD
      for c in range(3):
        plsc.store_scatter(outv, [b35 + c], plsc.load_gather(xv, [b3 + c]))
      ws = [wv[t, pl.ds(b, LANES)] for t in range(4 * NPL)]
      rvs = [bi + p * B for p in range(NPL)]
      for ch in range(FDIM_):
        acc = None
        for p in range(NPL):
          for q in range(4):
            cols = jnp.full((LANES,), q * FDIM_ + ch, jnp.int32)
            v = plsc.load_gather(rows, [rvs[p], cols]) * ws[p * 4 + q]
            acc = v if acc is None else acc + v
        plsc.store_scatter(outv, [b35 + (3 + ch)], acc)
      return 0

    lax.fori_loop(0, NV, acc_body, 0)
    pltpu.sync_copy(outv, out_hbm.at[pl.ds(base * OUTD, B * OUTD)])
    return 0

  lax.fori_loop(0, count, chunk_body, 0)


@jax.jit
def kernel(x, tgrid):
  tab = _build_quad_table(tgrid)
  mesh = plsc.VectorSubcoreMesh(core_axis_name="c", subcore_axis_name="s",
                                num_cores=NC, num_subcores=NS)
  run = pl.kernel(
      _tec_body,
      out_type=jax.ShapeDtypeStruct((NPTS * OUTD,), jnp.float32),
      mesh=mesh,
      compiler_params=pltpu.CompilerParams(needs_layout_passes=False,
                                           use_tc_tiling_on_sc=False),
      scratch_types=[
          pltpu.VMEM((B * 3,), jnp.float32),
          pltpu.VMEM((NPL, B), jnp.int32),
          pltpu.VMEM((4 * NPL, B), jnp.float32),
          pltpu.VMEM((NPL * B, QROW), jnp.float32),
          pltpu.VMEM((B * OUTD,), jnp.float32),
          pltpu.SemaphoreType.DMA,
      ],
  )
  out_flat = run(x.reshape(NPTS * 3), tab)
  return out_flat.reshape(NPTS, OUTD)
